# baseline (speedup 1.0000x reference)
import functools

import jax
import jax.numpy as jnp
from jax import lax
from jax.experimental import pallas as pl
from jax.experimental.pallas import tpu as pltpu
from jax.experimental.pallas import tpu_sc as plsc

_N = 65536
_D = 8
_TF = 65536.0
_INV_TF = 1.0 / 65536.0
_CM = 65535.0
_LANES = 16
_CHUNKS = 4
_CH = _N // _CHUNKS
_NV = _CH // _LANES


def _interval_contrib(lt, rt, c):
    dcm = c - _CM
    flat = dcm * dcm * (jnp.maximum(rt, _TF) - jnp.maximum(lt, _TF))
    l1 = jnp.minimum(lt, _TF)
    r1 = jnp.minimum(rt, _TF)
    p = l1.astype(jnp.int32).astype(jnp.float32)
    q = r1.astype(jnp.int32).astype(jnp.float32)
    cp = c - p
    cq = c - q
    g_same = (r1 - l1) * cp * cp
    n = q - 1.0 - p
    mu = (p + q) * 0.5
    cmu = c - mu
    g_diff = (
        (p + 1.0 - l1) * cp * cp
        + n * cmu * cmu
        + (n * n * n - n) * (1.0 / 12.0)
        + (r1 - q) * cq * cq
    )
    stair = jnp.where(p == q, g_same, g_diff)
    return flat + stair


def _make_sc_kernel(interpret=False):
    mesh = plsc.VectorSubcoreMesh(core_axis_name="c", subcore_axis_name="s")

    @functools.partial(
        pl.kernel,
        out_type=jax.ShapeDtypeStruct((_D * _CHUNKS, _LANES), jnp.float32),
        mesh=mesh,
        scratch_types=[
            pltpu.VMEM((_CH,), jnp.float32),
            pltpu.VMEM((_LANES,), jnp.float32),
            pltpu.VMEM((_LANES,), jnp.float32),
            pltpu.VMEM((_LANES, _LANES), jnp.float32),
            pltpu.VMEM_SHARED((_LANES, _LANES), jnp.float32),
            pltpu.SemaphoreType.DMA,
            pltpu.SemaphoreType.DMA,
        ],
        compiler_params=pltpu.CompilerParams(needs_layout_passes=False),
        interpret=interpret,
    )
    def sc_loss(x_hbm, out_hbm, xv, outv, sumv, allsums, sums_sh, sem_a, sem_b):
        cid = lax.axis_index("c")
        sid = lax.axis_index("s")
        grp_in_core = sid // _CHUNKS
        g = cid * _CHUNKS + grp_in_core
        k = sid % _CHUNKS
        row = g * _CHUNKS + k

        half = _CH // 2
        cp_a = pltpu.async_copy(
            x_hbm.at[row, pl.ds(0, half)], xv.at[pl.ds(0, half)], sem_a
        )
        cp_b = pltpu.async_copy(
            x_hbm.at[row, pl.ds(half, half)], xv.at[pl.ds(half, half)], sem_b
        )

        lane = lax.iota(jnp.int32, 16)
        lane_f = lane.astype(jnp.float32)
        c0 = (k * _CH).astype(jnp.float32) + lane_f
        zeros = jnp.zeros((_LANES,), jnp.float32)

        def body(i, carry):
            dcv_a, dcv_b, acc_a, acc_b, sums_a, sums_b = carry
            xa = xv[pl.ds(i * 32, 16)]
            xb = xv[pl.ds(i * 32 + 16, 16)]
            acc_a = acc_a + dcv_a * dcv_a * xa
            acc_b = acc_b + dcv_b * dcv_b * xb
            sums_a = sums_a + xa
            sums_b = sums_b + xb
            dcv_a = dcv_a + 32.0
            dcv_b = dcv_b + 32.0
            return dcv_a, dcv_b, acc_a, acc_b, sums_a, sums_b

        dc0 = c0 - _CM
        cp_a.wait()
        carry = lax.fori_loop(
            0, _NV // 4, body, (dc0, dc0 + 16.0, zeros, zeros, zeros, zeros),
            unroll=8,
        )
        cp_b.wait()
        _, _, acc_a, acc_b, sums_a, sums_b = lax.fori_loop(
            _NV // 4, _NV // 2, body, carry, unroll=8
        )
        acc = acc_a + acc_b
        chunk_sum = jnp.sum(sums_a + sums_b)

        sumv[...] = zeros + chunk_sum
        pltpu.sync_copy(sumv, sums_sh.at[sid])
        plsc.subcore_barrier()
        pltpu.sync_copy(sums_sh, allsums)

        base = grp_in_core * _CHUNKS
        offv = zeros
        for i in range(_CHUNKS - 1):
            offv = offv + jnp.where(i < k, allsums[base + i, :], zeros)
        off = offv[0]

        def cond(carry):
            i, run, _ = carry
            return jnp.logical_and(i < _NV, off + run < 1.0)

        def body2(carry):
            i, run, corr = carry
            x16 = xv[pl.ds(i * 16, 16)]
            pre = plsc.cumsum(x16)
            a16 = (off + run) + pre
            rt = a16 * _TF
            lt = (a16 - x16) * _TF
            c = c0 + (i * 16).astype(jnp.float32)
            dcm = c - _CM
            true_c = _interval_contrib(lt, rt, c) * _INV_TF
            fast_c = dcm * dcm * x16
            corr = corr + (true_c - fast_c)
            run = run + jnp.sum(x16)
            return i + 1, run, corr

        _, _, corr = lax.while_loop(
            cond, body2, (jnp.int32(0), jnp.float32(0.0), zeros)
        )
        acc = acc + corr

        @pl.when(k == _CHUNKS - 1)
        def _():
            ltail = (off + chunk_sum) * _TF
            tail = _interval_contrib(ltail, jnp.maximum(ltail, _TF), _CM)
            outv[...] = acc + jnp.where(lane == 0, tail * _INV_TF, 0.0)

        @pl.when(k != _CHUNKS - 1)
        def _():
            outv[...] = acc

        pltpu.sync_copy(outv, out_hbm.at[row])

    return sc_loss


@functools.lru_cache(maxsize=None)
def _get_sc_loss():
    return _make_sc_kernel()


def kernel(x, bins, bary_est):
    xs = x.reshape(_D * _CHUNKS, _CH)
    parts = _get_sc_loss()(xs)
    loss = jnp.sum(parts, dtype=jnp.float32).reshape(1)
    return (loss, bary_est)


# speedup vs baseline: 1.0176x; 1.0176x over previous
import functools

import jax
import jax.numpy as jnp
from jax import lax
from jax.experimental import pallas as pl
from jax.experimental.pallas import tpu as pltpu
from jax.experimental.pallas import tpu_sc as plsc

_N = 65536
_D = 8
_TF = 65536.0
_INV_TF = 1.0 / 65536.0
_CM = 65535.0
_LANES = 16
_CHUNKS = 4
_CH = _N // _CHUNKS
_NV = _CH // _LANES


def _interval_contrib(lt, rt, c):
    dcm = c - _CM
    flat = dcm * dcm * (jnp.maximum(rt, _TF) - jnp.maximum(lt, _TF))
    l1 = jnp.minimum(lt, _TF)
    r1 = jnp.minimum(rt, _TF)
    p = l1.astype(jnp.int32).astype(jnp.float32)
    q = r1.astype(jnp.int32).astype(jnp.float32)
    cp = c - p
    cq = c - q
    g_same = (r1 - l1) * cp * cp
    n = q - 1.0 - p
    mu = (p + q) * 0.5
    cmu = c - mu
    g_diff = (
        (p + 1.0 - l1) * cp * cp
        + n * cmu * cmu
        + (n * n * n - n) * (1.0 / 12.0)
        + (r1 - q) * cq * cq
    )
    stair = jnp.where(p == q, g_same, g_diff)
    return flat + stair


def _make_sc_kernel(interpret=False):
    mesh = plsc.VectorSubcoreMesh(core_axis_name="c", subcore_axis_name="s")

    @functools.partial(
        pl.kernel,
        out_type=jax.ShapeDtypeStruct((_D * _CHUNKS, _LANES), jnp.float32),
        mesh=mesh,
        scratch_types=[
            pltpu.VMEM((_CH,), jnp.float32),
            pltpu.VMEM((_LANES,), jnp.float32),
            pltpu.SemaphoreType.DMA,
            pltpu.SemaphoreType.DMA,
        ],
        compiler_params=pltpu.CompilerParams(needs_layout_passes=False),
        interpret=interpret,
    )
    def sc_loss(x_hbm, out_hbm, xv, outv, sem_a, sem_b):
        cid = lax.axis_index("c")
        sid = lax.axis_index("s")
        g = cid * _CHUNKS + sid // _CHUNKS
        k = sid % _CHUNKS
        row = g * _CHUNKS + k

        half = _CH // 2
        cp_a = pltpu.async_copy(
            x_hbm.at[row, pl.ds(0, half)], xv.at[pl.ds(0, half)], sem_a
        )
        cp_b = pltpu.async_copy(
            x_hbm.at[row, pl.ds(half, half)], xv.at[pl.ds(half, half)], sem_b
        )

        lane = lax.iota(jnp.int32, 16)
        lane_f = lane.astype(jnp.float32)
        c0 = (k * _CH).astype(jnp.float32) + lane_f
        zeros = jnp.zeros((_LANES,), jnp.float32)

        def body(i, carry):
            dcv_a, dcv_b, acc_a, acc_b = carry
            xa = xv[pl.ds(i * 32, 16)]
            xb = xv[pl.ds(i * 32 + 16, 16)]
            acc_a = acc_a + dcv_a * dcv_a * xa
            acc_b = acc_b + dcv_b * dcv_b * xb
            dcv_a = dcv_a + 32.0
            dcv_b = dcv_b + 32.0
            return dcv_a, dcv_b, acc_a, acc_b

        dc0 = c0 - _CM
        cp_a.wait()
        carry = lax.fori_loop(
            0, _NV // 4, body, (dc0, dc0 + 16.0, zeros, zeros), unroll=8
        )
        cp_b.wait()
        _, _, acc_a, acc_b = lax.fori_loop(
            _NV // 4, _NV // 2, body, carry, unroll=8
        )
        acc = acc_a + acc_b

        @pl.when(k == 0)
        def _():
            def walk_chunk(kk, run, corr):
                ck0 = (kk * _CH).astype(jnp.float32) + lane_f

                def cond(carry):
                    i, run, _ = carry
                    return jnp.logical_and(i < _NV, run < 1.0)

                def body2(carry):
                    i, run, corr = carry
                    x16 = xv[pl.ds(i * 16, 16)]
                    pre = plsc.cumsum(x16)
                    a16 = run + pre
                    rt = a16 * _TF
                    lt = (a16 - x16) * _TF
                    c = ck0 + (i * 16).astype(jnp.float32)
                    dcm = c - _CM
                    true_c = _interval_contrib(lt, rt, c) * _INV_TF
                    fast_c = dcm * dcm * x16
                    corr = corr + (true_c - fast_c)
                    run = run + jnp.sum(x16)
                    return i + 1, run, corr

                return lax.while_loop(cond, body2, (jnp.int32(0), run, corr))

            _, run, corr = walk_chunk(jnp.int32(0), jnp.float32(0.0), zeros)
            for kk in range(1, _CHUNKS):

                def cont(operands, kk=kk):
                    run, corr = operands
                    pltpu.sync_copy(x_hbm.at[g * _CHUNKS + kk], xv)
                    _, run, corr = walk_chunk(jnp.int32(kk), run, corr)
                    return run, corr

                run, corr = lax.cond(
                    run < 1.0, cont, lambda o: o, (run, corr)
                )

            ltail = run * _TF
            tail = _interval_contrib(ltail, jnp.maximum(ltail, _TF), _CM)
            outv[...] = (acc + corr) + jnp.where(lane == 0, tail * _INV_TF, 0.0)

        @pl.when(k != 0)
        def _():
            outv[...] = acc

        pltpu.sync_copy(outv, out_hbm.at[row])

    return sc_loss


@functools.lru_cache(maxsize=None)
def _get_sc_loss():
    return _make_sc_kernel()


def kernel(x, bins, bary_est):
    xs = x.reshape(_D * _CHUNKS, _CH)
    parts = _get_sc_loss()(xs)
    loss = jnp.sum(parts, dtype=jnp.float32).reshape(1)
    return (loss, bary_est)
